# baseline (speedup 1.0000x reference)
import functools

import jax
import jax.numpy as jnp
from jax import lax
from jax.experimental import pallas as pl
from jax.experimental.pallas import tpu as pltpu
from jax.experimental.pallas import tpu_sc as plsc

HIDDEN = 2048
FFN = 4096
NUM_EXPERTS = 8
TOP_K = 2
SEQ = 2048

M_BLK = 256
NV = SEQ * TOP_K // M_BLK + (NUM_EXPERTS - 1)
M_PAD = NV * M_BLK
N_BLK = 1024
NN = FFN // N_BLK


def _router_body(hs_ref, gate_ref, logits_ref, ids_ref, wts_ref):
    logits = jnp.dot(hs_ref[...], gate_ref[...],
                     preferred_element_type=jnp.float32)
    logits_ref[...] = logits
    col = lax.broadcasted_iota(jnp.int32, logits.shape, 1)
    m1 = jnp.max(logits, axis=1, keepdims=True)
    a1 = jnp.min(jnp.where(logits == m1, col, NUM_EXPERTS), axis=1,
                 keepdims=True)
    masked = jnp.where(col == a1, -jnp.inf, logits)
    m2 = jnp.max(masked, axis=1, keepdims=True)
    a2 = jnp.min(jnp.where(masked == m2, col, NUM_EXPERTS), axis=1,
                 keepdims=True)
    z = jnp.exp(m2 - m1)
    w2_ = z / (1.0 + z)
    w1_ = 1.0 / (1.0 + z)
    ids_ref[...] = jnp.concatenate([a1, a2], axis=1)
    wts_ref[...] = jnp.concatenate([w1_, w2_], axis=1)


def _router(hs2d, gate_w):
    return pl.pallas_call(
        _router_body,
        out_shape=(
            jax.ShapeDtypeStruct((SEQ, NUM_EXPERTS), jnp.float32),
            jax.ShapeDtypeStruct((SEQ, TOP_K), jnp.int32),
            jax.ShapeDtypeStruct((SEQ, TOP_K), jnp.float32),
        ),
    )(hs2d, gate_w)


def _k1_body(meta_ref, x_ref, w1_ref, w3_ref, h_ref):
    x = x_ref[...].astype(jnp.bfloat16)
    h1 = jnp.dot(x, w1_ref[0].astype(jnp.bfloat16),
                 preferred_element_type=jnp.float32)
    h3 = jnp.dot(x, w3_ref[0].astype(jnp.bfloat16),
                 preferred_element_type=jnp.float32)
    h_ref[...] = ((h1 * jax.nn.sigmoid(h1)) * h3).astype(jnp.bfloat16)


def _k1(x_pad, w1, w3, tile_expert):
    grid_spec = pltpu.PrefetchScalarGridSpec(
        num_scalar_prefetch=1,
        grid=(NN, NV),
        in_specs=[
            pl.BlockSpec((M_BLK, HIDDEN), lambda n, v, m: (v, 0)),
            pl.BlockSpec((1, HIDDEN, N_BLK), lambda n, v, m: (m[v], 0, n)),
            pl.BlockSpec((1, HIDDEN, N_BLK), lambda n, v, m: (m[v], 0, n)),
        ],
        out_specs=pl.BlockSpec((M_BLK, N_BLK), lambda n, v, m: (v, n)),
    )
    return pl.pallas_call(
        _k1_body,
        grid_spec=grid_spec,
        out_shape=jax.ShapeDtypeStruct((M_PAD, FFN), jnp.bfloat16),
        compiler_params=pltpu.CompilerParams(
            dimension_semantics=("arbitrary", "arbitrary"),
        ),
    )(tile_expert, x_pad, w1, w3)


N2_BLK = 2048
NK2 = FFN // N2_BLK


def _k2_body(meta_ref, h_ref, w2_ref, wcol_ref, y_ref):
    y_ref[...] = jnp.dot(h_ref[...], w2_ref[0].astype(jnp.bfloat16),
                         preferred_element_type=jnp.float32) * wcol_ref[...]


def _k2(h, w2, w_col, tile_expert):
    grid_spec = pltpu.PrefetchScalarGridSpec(
        num_scalar_prefetch=1,
        grid=(NK2, NV),
        in_specs=[
            pl.BlockSpec((M_BLK, N2_BLK), lambda k, v, m: (v, k)),
            pl.BlockSpec((1, N2_BLK, HIDDEN), lambda k, v, m: (m[v], k, 0)),
            pl.BlockSpec((M_BLK, 1), lambda k, v, m: (v, 0)),
        ],
        out_specs=pl.BlockSpec((M_BLK, HIDDEN), lambda k, v, m: (k * NV + v, 0)),
    )
    return pl.pallas_call(
        _k2_body,
        grid_spec=grid_spec,
        out_shape=jax.ShapeDtypeStruct((NK2 * M_PAD, HIDDEN), jnp.float32),
        compiler_params=pltpu.CompilerParams(
            dimension_semantics=("arbitrary", "arbitrary"),
        ),
    )(tile_expert, h, w2, w_col)


_SC_INFO = plsc.get_sparse_core_info()
_NC, _NS = _SC_INFO.num_cores, _SC_INFO.num_subcores
_NW = _NC * _NS
_G_CHUNK = 8
_G_ROWS_PER_W = M_PAD // _NW
_G_CHUNKS = _G_ROWS_PER_W // _G_CHUNK
_C_CHUNK = 8
_C_TOK_PER_W = SEQ // _NW
_C_CHUNKS = _C_TOK_PER_W // _C_CHUNK
_SC_MESH = plsc.VectorSubcoreMesh(core_axis_name="c", subcore_axis_name="s")


@functools.partial(
    pl.kernel, mesh=_SC_MESH,
    out_type=jax.ShapeDtypeStruct((M_PAD, HIDDEN), jnp.float32),
    scratch_types=[
        pltpu.VMEM((2, _G_CHUNK), jnp.int32),
        pltpu.VMEM((2, _G_CHUNK, HIDDEN), jnp.float32),
        pltpu.SemaphoreType.DMA,
        pltpu.SemaphoreType.DMA,
    ],
)
def _sc_gather(hs_hbm, idx_hbm, out_hbm, idx_v, rows_v, sem0, sem1):
    wid = lax.axis_index("s") * _NC + lax.axis_index("c")
    base0 = wid * _G_ROWS_PER_W
    sems = (sem0, sem1)
    pltpu.sync_copy(idx_hbm.at[pl.ds(base0, _G_CHUNK)], idx_v.at[0])
    cp = pltpu.async_copy(hs_hbm.at[idx_v.at[0]], rows_v.at[0], sems[0])
    copies = [cp]
    for i in range(_G_CHUNKS):
        b = i % 2
        if i + 1 < _G_CHUNKS:
            nb = (i + 1) % 2
            pltpu.sync_copy(
                idx_hbm.at[pl.ds(base0 + (i + 1) * _G_CHUNK, _G_CHUNK)],
                idx_v.at[nb])
            copies.append(pltpu.async_copy(
                hs_hbm.at[idx_v.at[nb]], rows_v.at[nb], sems[nb]))
        copies[i].wait()
        pltpu.sync_copy(rows_v.at[b],
                        out_hbm.at[pl.ds(base0 + i * _G_CHUNK, _G_CHUNK)])


@functools.partial(
    pl.kernel, mesh=_SC_MESH,
    out_type=jax.ShapeDtypeStruct((SEQ, HIDDEN), jnp.float32),
    scratch_types=[
        pltpu.VMEM((4, _C_CHUNK), jnp.int32),
        pltpu.VMEM((4, _C_CHUNK, HIDDEN), jnp.float32),
        pltpu.VMEM((_C_CHUNK, HIDDEN), jnp.float32),
        pltpu.SemaphoreType.DMA,
    ],
)
def _sc_combine(y_hbm, pidx_hbm, out_hbm, idx_v, bufs_v, acc_v, sem):
    wid = lax.axis_index("s") * _NC + lax.axis_index("c")
    base0 = wid * _C_TOK_PER_W
    for i in range(_C_CHUNKS):
        base = base0 + i * _C_CHUNK
        copies = []
        for s in range(4):
            pltpu.sync_copy(pidx_hbm.at[s, pl.ds(base, _C_CHUNK)],
                            idx_v.at[s])
            copies.append(pltpu.async_copy(
                y_hbm.at[idx_v.at[s]], bufs_v.at[s], sem))
        for cp in copies:
            cp.wait()

        def add_body(j, _):
            r = j // (HIDDEN // 16)
            off = (j % (HIDDEN // 16)) * 16
            acc_v[r, pl.ds(off, 16)] = (
                (bufs_v[0, r, pl.ds(off, 16)] + bufs_v[1, r, pl.ds(off, 16)])
                + (bufs_v[2, r, pl.ds(off, 16)]
                   + bufs_v[3, r, pl.ds(off, 16)]))
            return 0

        lax.fori_loop(0, _C_CHUNK * (HIDDEN // 16), add_body, 0, unroll=4)
        pltpu.sync_copy(acc_v, out_hbm.at[pl.ds(base, _C_CHUNK)])


def kernel(hidden_states, gate_w, w1, w2, w3):
    hs2d = hidden_states.reshape(-1, HIDDEN)

    router_logits, ids, wts = _router(hs2d, gate_w)

    e_flat = ids.reshape(-1)
    onehot = (e_flat[:, None] == jnp.arange(NUM_EXPERTS)[None, :]).astype(
        jnp.int32)
    csum = jnp.cumsum(onehot, axis=0)
    rank = jnp.take_along_axis(csum, e_flat[:, None], axis=1)[:, 0] - 1
    counts = csum[-1]
    tiles_e = (counts + M_BLK - 1) // M_BLK
    tile_start = jnp.concatenate(
        [jnp.zeros((1,), jnp.int32), jnp.cumsum(tiles_e)[:-1]])
    pos = tile_start[e_flat] * M_BLK + rank
    src_row = jnp.zeros((M_PAD,), jnp.int32).at[pos].set(
        jnp.arange(SEQ * TOP_K, dtype=jnp.int32) // TOP_K)
    w_col = jnp.zeros((M_PAD,), jnp.float32).at[pos].set(
        wts.reshape(-1)).reshape(M_PAD, 1)
    tile_expert = jnp.minimum(
        jnp.searchsorted(jnp.cumsum(tiles_e), jnp.arange(NV), side="right"),
        NUM_EXPERTS - 1).astype(jnp.int32)

    x_pad = _sc_gather(hs2d, src_row)

    h = _k1(x_pad, w1, w3, tile_expert)
    yw = _k2(h, w2, w_col, tile_expert)

    p = pos.reshape(SEQ, TOP_K)
    pidx = jnp.stack([p[:, 0], p[:, 0] + M_PAD,
                      p[:, 1], p[:, 1] + M_PAD]).astype(jnp.int32)
    final = _sc_combine(yw, pidx)
    return final, router_logits


# speedup vs baseline: 1.0500x; 1.0500x over previous
import functools

import jax
import jax.numpy as jnp
from jax import lax
from jax.experimental import pallas as pl
from jax.experimental.pallas import tpu as pltpu
from jax.experimental.pallas import tpu_sc as plsc

HIDDEN = 2048
FFN = 4096
NUM_EXPERTS = 8
TOP_K = 2
SEQ = 2048

M_BLK = 256
NV = SEQ * TOP_K // M_BLK + (NUM_EXPERTS - 1)
M_PAD = NV * M_BLK
N_BLK = 1024
NN = FFN // N_BLK


def _router_body(hs_ref, gate_ref, logits_ref, ids_ref, wts_ref):
    logits = jnp.dot(hs_ref[...], gate_ref[...],
                     preferred_element_type=jnp.float32)
    logits_ref[...] = logits
    col = lax.broadcasted_iota(jnp.int32, logits.shape, 1)
    m1 = jnp.max(logits, axis=1, keepdims=True)
    a1 = jnp.min(jnp.where(logits == m1, col, NUM_EXPERTS), axis=1,
                 keepdims=True)
    masked = jnp.where(col == a1, -jnp.inf, logits)
    m2 = jnp.max(masked, axis=1, keepdims=True)
    a2 = jnp.min(jnp.where(masked == m2, col, NUM_EXPERTS), axis=1,
                 keepdims=True)
    z = jnp.exp(m2 - m1)
    w2_ = z / (1.0 + z)
    w1_ = 1.0 / (1.0 + z)
    ids_ref[...] = jnp.concatenate([a1, a2], axis=1)
    wts_ref[...] = jnp.concatenate([w1_, w2_], axis=1)


def _router(hs2d, gate_w):
    return pl.pallas_call(
        _router_body,
        out_shape=(
            jax.ShapeDtypeStruct((SEQ, NUM_EXPERTS), jnp.float32),
            jax.ShapeDtypeStruct((SEQ, TOP_K), jnp.int32),
            jax.ShapeDtypeStruct((SEQ, TOP_K), jnp.float32),
        ),
    )(hs2d, gate_w)


def _k1_body(meta_ref, x_ref, w1_ref, w3_ref, h_ref):
    x = x_ref[...].astype(jnp.bfloat16)
    h1 = jnp.dot(x, w1_ref[0].astype(jnp.bfloat16),
                 preferred_element_type=jnp.float32)
    h3 = jnp.dot(x, w3_ref[0].astype(jnp.bfloat16),
                 preferred_element_type=jnp.float32)
    h_ref[...] = ((h1 * jax.nn.sigmoid(h1)) * h3).astype(jnp.bfloat16)


def _k1(x_pad, w1, w3, tile_expert):
    grid_spec = pltpu.PrefetchScalarGridSpec(
        num_scalar_prefetch=1,
        grid=(NN, NV),
        in_specs=[
            pl.BlockSpec((M_BLK, HIDDEN), lambda n, v, m: (v, 0)),
            pl.BlockSpec((1, HIDDEN, N_BLK), lambda n, v, m: (m[v], 0, n)),
            pl.BlockSpec((1, HIDDEN, N_BLK), lambda n, v, m: (m[v], 0, n)),
        ],
        out_specs=pl.BlockSpec((M_BLK, N_BLK), lambda n, v, m: (v, n)),
    )
    return pl.pallas_call(
        _k1_body,
        grid_spec=grid_spec,
        out_shape=jax.ShapeDtypeStruct((M_PAD, FFN), jnp.bfloat16),
        compiler_params=pltpu.CompilerParams(
            dimension_semantics=("arbitrary", "arbitrary"),
        ),
    )(tile_expert, x_pad, w1, w3)


H2_BLK = 1024
NH2 = HIDDEN // H2_BLK


def _k2_body(meta_ref, h_ref, w2_ref, wcol_ref, y_ref):
    y_ref[...] = jnp.dot(h_ref[...], w2_ref[0].astype(jnp.bfloat16),
                         preferred_element_type=jnp.float32) * wcol_ref[...]


def _k2(h, w2, w_col, tile_expert):
    grid_spec = pltpu.PrefetchScalarGridSpec(
        num_scalar_prefetch=1,
        grid=(NH2, NV),
        in_specs=[
            pl.BlockSpec((M_BLK, FFN), lambda c, v, m: (v, 0)),
            pl.BlockSpec((1, FFN, H2_BLK), lambda c, v, m: (m[v], 0, c)),
            pl.BlockSpec((M_BLK, 1), lambda c, v, m: (v, 0)),
        ],
        out_specs=pl.BlockSpec((M_BLK, H2_BLK), lambda c, v, m: (v, c)),
    )
    return pl.pallas_call(
        _k2_body,
        grid_spec=grid_spec,
        out_shape=jax.ShapeDtypeStruct((M_PAD, HIDDEN), jnp.float32),
        compiler_params=pltpu.CompilerParams(
            dimension_semantics=("arbitrary", "arbitrary"),
        ),
    )(tile_expert, h, w2, w_col)


_SC_INFO = plsc.get_sparse_core_info()
_NC, _NS = _SC_INFO.num_cores, _SC_INFO.num_subcores
_NW = _NC * _NS
_G_CHUNK = 8
_G_ROWS_PER_W = M_PAD // _NW
_G_CHUNKS = _G_ROWS_PER_W // _G_CHUNK
_C_CHUNK = 8
_C_TOK_PER_W = SEQ // _NW
_C_CHUNKS = _C_TOK_PER_W // _C_CHUNK
_SC_MESH = plsc.VectorSubcoreMesh(core_axis_name="c", subcore_axis_name="s")


@functools.partial(
    pl.kernel, mesh=_SC_MESH,
    out_type=jax.ShapeDtypeStruct((M_PAD, HIDDEN), jnp.float32),
    scratch_types=[
        pltpu.VMEM((2, _G_CHUNK), jnp.int32),
        pltpu.VMEM((2, _G_CHUNK, HIDDEN), jnp.float32),
        pltpu.SemaphoreType.DMA,
        pltpu.SemaphoreType.DMA,
    ],
)
def _sc_gather(hs_hbm, idx_hbm, out_hbm, idx_v, rows_v, sem0, sem1):
    wid = lax.axis_index("s") * _NC + lax.axis_index("c")
    base0 = wid * _G_ROWS_PER_W
    sems = (sem0, sem1)
    pltpu.sync_copy(idx_hbm.at[pl.ds(base0, _G_CHUNK)], idx_v.at[0])
    cp = pltpu.async_copy(hs_hbm.at[idx_v.at[0]], rows_v.at[0], sems[0])
    copies = [cp]
    for i in range(_G_CHUNKS):
        b = i % 2
        if i + 1 < _G_CHUNKS:
            nb = (i + 1) % 2
            pltpu.sync_copy(
                idx_hbm.at[pl.ds(base0 + (i + 1) * _G_CHUNK, _G_CHUNK)],
                idx_v.at[nb])
            copies.append(pltpu.async_copy(
                hs_hbm.at[idx_v.at[nb]], rows_v.at[nb], sems[nb]))
        copies[i].wait()
        pltpu.sync_copy(rows_v.at[b],
                        out_hbm.at[pl.ds(base0 + i * _G_CHUNK, _G_CHUNK)])


@functools.partial(
    pl.kernel, mesh=_SC_MESH,
    out_type=jax.ShapeDtypeStruct((SEQ, HIDDEN), jnp.float32),
    scratch_types=[
        pltpu.VMEM((2, _C_CHUNK), jnp.int32),
        pltpu.VMEM((2, _C_CHUNK, HIDDEN), jnp.float32),
        pltpu.VMEM((_C_CHUNK, HIDDEN), jnp.float32),
        pltpu.SemaphoreType.DMA,
    ],
)
def _sc_combine(y_hbm, pidx_hbm, out_hbm, idx_v, bufs_v, acc_v, sem):
    wid = lax.axis_index("s") * _NC + lax.axis_index("c")
    base0 = wid * _C_TOK_PER_W
    for i in range(_C_CHUNKS):
        base = base0 + i * _C_CHUNK
        copies = []
        for s in range(2):
            pltpu.sync_copy(pidx_hbm.at[s, pl.ds(base, _C_CHUNK)],
                            idx_v.at[s])
            copies.append(pltpu.async_copy(
                y_hbm.at[idx_v.at[s]], bufs_v.at[s], sem))
        for cp in copies:
            cp.wait()

        def add_body(j, _):
            r = j // (HIDDEN // 16)
            off = (j % (HIDDEN // 16)) * 16
            acc_v[r, pl.ds(off, 16)] = (
                bufs_v[0, r, pl.ds(off, 16)] + bufs_v[1, r, pl.ds(off, 16)])
            return 0

        lax.fori_loop(0, _C_CHUNK * (HIDDEN // 16), add_body, 0, unroll=4)
        pltpu.sync_copy(acc_v, out_hbm.at[pl.ds(base, _C_CHUNK)])


def kernel(hidden_states, gate_w, w1, w2, w3):
    hs2d = hidden_states.reshape(-1, HIDDEN)

    router_logits, ids, wts = _router(hs2d, gate_w)

    e_flat = ids.reshape(-1)
    onehot = (e_flat[:, None] == jnp.arange(NUM_EXPERTS)[None, :]).astype(
        jnp.int32)
    csum = jnp.cumsum(onehot, axis=0)
    rank = jnp.take_along_axis(csum, e_flat[:, None], axis=1)[:, 0] - 1
    counts = csum[-1]
    tiles_e = (counts + M_BLK - 1) // M_BLK
    tile_start = jnp.concatenate(
        [jnp.zeros((1,), jnp.int32), jnp.cumsum(tiles_e)[:-1]])
    pos = tile_start[e_flat] * M_BLK + rank
    src_row = jnp.zeros((M_PAD,), jnp.int32).at[pos].set(
        jnp.arange(SEQ * TOP_K, dtype=jnp.int32) // TOP_K)
    w_col = jnp.zeros((M_PAD,), jnp.float32).at[pos].set(
        wts.reshape(-1)).reshape(M_PAD, 1)
    tile_expert = jnp.minimum(
        jnp.searchsorted(jnp.cumsum(tiles_e), jnp.arange(NV), side="right"),
        NUM_EXPERTS - 1).astype(jnp.int32)

    x_pad = _sc_gather(hs2d, src_row)

    h = _k1(x_pad, w1, w3, tile_expert)
    yw = _k2(h, w2, w_col, tile_expert)

    p = pos.reshape(SEQ, TOP_K)
    pidx = jnp.stack([p[:, 0], p[:, 1]]).astype(jnp.int32)
    final = _sc_combine(yw, pidx)
    return final, router_logits


# speedup vs baseline: 1.0626x; 1.0120x over previous
import functools

import jax
import jax.numpy as jnp
from jax import lax
from jax.experimental import pallas as pl
from jax.experimental.pallas import tpu as pltpu
from jax.experimental.pallas import tpu_sc as plsc

HIDDEN = 2048
FFN = 4096
NUM_EXPERTS = 8
TOP_K = 2
SEQ = 2048

M_BLK = 256
NV = SEQ * TOP_K // M_BLK + (NUM_EXPERTS - 1)
M_PAD = NV * M_BLK
N_BLK = 1024
NN = FFN // N_BLK


def _router_body(hs_ref, gate_ref, logits_ref, ids_ref, wts_ref):
    logits = jnp.dot(hs_ref[...], gate_ref[...],
                     preferred_element_type=jnp.float32)
    logits_ref[...] = logits
    col = lax.broadcasted_iota(jnp.int32, logits.shape, 1)
    m1 = jnp.max(logits, axis=1, keepdims=True)
    a1 = jnp.min(jnp.where(logits == m1, col, NUM_EXPERTS), axis=1,
                 keepdims=True)
    masked = jnp.where(col == a1, -jnp.inf, logits)
    m2 = jnp.max(masked, axis=1, keepdims=True)
    a2 = jnp.min(jnp.where(masked == m2, col, NUM_EXPERTS), axis=1,
                 keepdims=True)
    z = jnp.exp(m2 - m1)
    w2_ = z / (1.0 + z)
    w1_ = 1.0 / (1.0 + z)
    ids_ref[...] = jnp.concatenate([a1, a2], axis=1)
    wts_ref[...] = jnp.concatenate([w1_, w2_], axis=1)


def _router(hs2d, gate_w):
    return pl.pallas_call(
        _router_body,
        out_shape=(
            jax.ShapeDtypeStruct((SEQ, NUM_EXPERTS), jnp.float32),
            jax.ShapeDtypeStruct((SEQ, TOP_K), jnp.int32),
            jax.ShapeDtypeStruct((SEQ, TOP_K), jnp.float32),
        ),
    )(hs2d, gate_w)


def _k1_body(meta_ref, x_ref, w1_ref, w3_ref, h_ref):
    x = x_ref[...].astype(jnp.bfloat16)
    h1 = jnp.dot(x, w1_ref[0].astype(jnp.bfloat16),
                 preferred_element_type=jnp.float32)
    h3 = jnp.dot(x, w3_ref[0].astype(jnp.bfloat16),
                 preferred_element_type=jnp.float32)
    h_ref[...] = ((h1 * jax.nn.sigmoid(h1)) * h3).astype(jnp.bfloat16)


def _k1(x_pad, w1, w3, tile_expert):
    grid_spec = pltpu.PrefetchScalarGridSpec(
        num_scalar_prefetch=1,
        grid=(NN, NV),
        in_specs=[
            pl.BlockSpec((M_BLK, HIDDEN), lambda n, v, m: (v, 0)),
            pl.BlockSpec((1, HIDDEN, N_BLK), lambda n, v, m: (m[v], 0, n)),
            pl.BlockSpec((1, HIDDEN, N_BLK), lambda n, v, m: (m[v], 0, n)),
        ],
        out_specs=pl.BlockSpec((M_BLK, N_BLK), lambda n, v, m: (v, n)),
    )
    return pl.pallas_call(
        _k1_body,
        grid_spec=grid_spec,
        out_shape=jax.ShapeDtypeStruct((M_PAD, FFN), jnp.bfloat16),
        compiler_params=pltpu.CompilerParams(
            dimension_semantics=("arbitrary", "arbitrary"),
        ),
    )(tile_expert, x_pad, w1, w3)


H2_BLK = 1024
NH2 = HIDDEN // H2_BLK


def _k2_body(meta_ref, h_ref, w2_ref, wcol_ref, y_ref):
    y_ref[...] = jnp.dot(h_ref[...], w2_ref[0].astype(jnp.bfloat16),
                         preferred_element_type=jnp.float32) * wcol_ref[...]


def _k2(h, w2, w_col, tile_expert):
    grid_spec = pltpu.PrefetchScalarGridSpec(
        num_scalar_prefetch=1,
        grid=(NH2, NV),
        in_specs=[
            pl.BlockSpec((M_BLK, FFN), lambda c, v, m: (v, 0)),
            pl.BlockSpec((1, FFN, H2_BLK), lambda c, v, m: (m[v], 0, c)),
            pl.BlockSpec((M_BLK, 1), lambda c, v, m: (v, 0)),
        ],
        out_specs=pl.BlockSpec((M_BLK, H2_BLK), lambda c, v, m: (v, c)),
    )
    return pl.pallas_call(
        _k2_body,
        grid_spec=grid_spec,
        out_shape=jax.ShapeDtypeStruct((M_PAD, HIDDEN), jnp.float32),
        compiler_params=pltpu.CompilerParams(
            dimension_semantics=("arbitrary", "arbitrary"),
        ),
    )(tile_expert, h, w2, w_col)


_SC_INFO = plsc.get_sparse_core_info()
_NC, _NS = _SC_INFO.num_cores, _SC_INFO.num_subcores
_NW = _NC * _NS
_G_CHUNK = 8
_G_ROWS_PER_W = M_PAD // _NW
_G_CHUNKS = _G_ROWS_PER_W // _G_CHUNK
_C_CHUNK = 8
_C_TOK_PER_W = SEQ // _NW
_C_CHUNKS = _C_TOK_PER_W // _C_CHUNK
_SC_MESH = plsc.VectorSubcoreMesh(core_axis_name="c", subcore_axis_name="s")


@functools.partial(
    pl.kernel, mesh=_SC_MESH,
    out_type=jax.ShapeDtypeStruct((M_PAD, HIDDEN), jnp.float32),
    scratch_types=[
        pltpu.VMEM((4, _G_CHUNK), jnp.int32),
        pltpu.VMEM((4, _G_CHUNK, HIDDEN), jnp.float32),
        pltpu.SemaphoreType.DMA,
        pltpu.SemaphoreType.DMA,
        pltpu.SemaphoreType.DMA,
        pltpu.SemaphoreType.DMA,
    ],
)
def _sc_gather(hs_hbm, idx_hbm, out_hbm, idx_v, rows_v, s0, s1, s2, s3):
    wid = lax.axis_index("s") * _NC + lax.axis_index("c")
    base0 = wid * _G_ROWS_PER_W
    sems = (s0, s1, s2, s3)
    depth = 4
    copies = []
    for i in range(min(depth, _G_CHUNKS)):
        pltpu.sync_copy(idx_hbm.at[pl.ds(base0 + i * _G_CHUNK, _G_CHUNK)],
                        idx_v.at[i])
        copies.append(pltpu.async_copy(
            hs_hbm.at[idx_v.at[i]], rows_v.at[i], sems[i]))
    for i in range(_G_CHUNKS):
        b = i % depth
        copies[i].wait()
        pltpu.sync_copy(rows_v.at[b],
                        out_hbm.at[pl.ds(base0 + i * _G_CHUNK, _G_CHUNK)])
        j = i + depth
        if j < _G_CHUNKS:
            pltpu.sync_copy(
                idx_hbm.at[pl.ds(base0 + j * _G_CHUNK, _G_CHUNK)],
                idx_v.at[b])
            copies.append(pltpu.async_copy(
                hs_hbm.at[idx_v.at[b]], rows_v.at[b], sems[b]))


@functools.partial(
    pl.kernel, mesh=_SC_MESH,
    out_type=jax.ShapeDtypeStruct((SEQ, HIDDEN), jnp.float32),
    scratch_types=[
        pltpu.VMEM((2, 2, _C_CHUNK), jnp.int32),
        pltpu.VMEM((2, 2, _C_CHUNK, HIDDEN), jnp.float32),
        pltpu.VMEM((_C_CHUNK, HIDDEN), jnp.float32),
        pltpu.SemaphoreType.DMA,
        pltpu.SemaphoreType.DMA,
    ],
)
def _sc_combine(y_hbm, pidx_hbm, out_hbm, idx_v, bufs_v, acc_v, sem0, sem1):
    wid = lax.axis_index("s") * _NC + lax.axis_index("c")
    base0 = wid * _C_TOK_PER_W
    sems = (sem0, sem1)

    def fire(i, b):
        for s in range(2):
            pltpu.sync_copy(pidx_hbm.at[s, pl.ds(base0 + i * _C_CHUNK,
                                                 _C_CHUNK)],
                            idx_v.at[b, s])
        return [pltpu.async_copy(y_hbm.at[idx_v.at[b, s]],
                                 bufs_v.at[b, s], sems[b])
                for s in range(2)]

    copies = [fire(0, 0)]
    for i in range(_C_CHUNKS):
        b = i % 2
        if i + 1 < _C_CHUNKS:
            copies.append(fire(i + 1, (i + 1) % 2))
        for cp in copies[i]:
            cp.wait()

        def add_body(j, _):
            r = j // (HIDDEN // 16)
            off = (j % (HIDDEN // 16)) * 16
            acc_v[r, pl.ds(off, 16)] = (
                bufs_v[b, 0, r, pl.ds(off, 16)]
                + bufs_v[b, 1, r, pl.ds(off, 16)])
            return 0

        lax.fori_loop(0, _C_CHUNK * (HIDDEN // 16), add_body, 0, unroll=4)
        pltpu.sync_copy(acc_v, out_hbm.at[pl.ds(base0 + i * _C_CHUNK,
                                                _C_CHUNK)])


def kernel(hidden_states, gate_w, w1, w2, w3):
    hs2d = hidden_states.reshape(-1, HIDDEN)

    router_logits, ids, wts = _router(hs2d, gate_w)

    e_flat = ids.reshape(-1)
    onehot = (e_flat[:, None] == jnp.arange(NUM_EXPERTS)[None, :]).astype(
        jnp.int32)
    csum = jnp.cumsum(onehot, axis=0)
    rank = jnp.take_along_axis(csum, e_flat[:, None], axis=1)[:, 0] - 1
    counts = csum[-1]
    tiles_e = (counts + M_BLK - 1) // M_BLK
    tile_start = jnp.concatenate(
        [jnp.zeros((1,), jnp.int32), jnp.cumsum(tiles_e)[:-1]])
    pos = tile_start[e_flat] * M_BLK + rank
    src_row = jnp.zeros((M_PAD,), jnp.int32).at[pos].set(
        jnp.arange(SEQ * TOP_K, dtype=jnp.int32) // TOP_K)
    w_col = jnp.zeros((M_PAD,), jnp.float32).at[pos].set(
        wts.reshape(-1)).reshape(M_PAD, 1)
    tile_expert = jnp.minimum(
        jnp.searchsorted(jnp.cumsum(tiles_e), jnp.arange(NV), side="right"),
        NUM_EXPERTS - 1).astype(jnp.int32)

    x_pad = _sc_gather(hs2d, src_row)

    h = _k1(x_pad, w1, w3, tile_expert)
    yw = _k2(h, w2, w_col, tile_expert)

    p = pos.reshape(SEQ, TOP_K)
    pidx = jnp.stack([p[:, 0], p[:, 1]]).astype(jnp.int32)
    final = _sc_combine(yw, pidx)
    return final, router_logits
